# baseline (speedup 1.0000x reference)
import functools

import jax
import jax.numpy as jnp
from jax import lax
from jax.experimental import pallas as pl
from jax.experimental.pallas import tpu as pltpu
from jax.experimental.pallas import tpu_sc as plsc

NUM_SCENES = 100000
NUM_LOBES = 128
NUM_P = 7
BATCH = 16384
CHUNK = 64


def _make_gather():
  info = plsc.get_sparse_core_info()
  nw = info.num_cores * info.num_subcores
  b_per_w = BATCH // nw
  n_chunks = b_per_w // CHUNK

  mesh = plsc.VectorSubcoreMesh(core_axis_name="c", subcore_axis_name="s")

  @functools.partial(
      pl.kernel,
      mesh=mesh,
      out_type=jax.ShapeDtypeStruct((NUM_P, BATCH, NUM_LOBES), jnp.float32),
      scratch_types=[
          pltpu.VMEM((b_per_w,), jnp.int32),
          pltpu.VMEM((CHUNK, NUM_LOBES), jnp.float32),
          pltpu.VMEM((CHUNK, NUM_LOBES), jnp.float32),
          pltpu.VMEM((CHUNK, NUM_LOBES), jnp.float32),
          pltpu.VMEM((CHUNK, NUM_LOBES), jnp.float32),
          pltpu.VMEM((CHUNK, NUM_LOBES), jnp.float32),
          pltpu.VMEM((CHUNK, NUM_LOBES), jnp.float32),
          pltpu.VMEM((CHUNK, NUM_LOBES), jnp.float32),
          pltpu.VMEM((CHUNK, NUM_LOBES), jnp.float32),
          pltpu.SemaphoreType.DMA,
          pltpu.SemaphoreType.DMA,
      ],
  )
  def gather_kernel(table_hbm, idx_hbm, out_hbm, idx_v, buf0, buf1,
                    buf2, buf3, buf4, buf5, buf6, buf7, gsem, wsem):
    wid = lax.axis_index("s") * info.num_cores + lax.axis_index("c")
    base = wid * b_per_w
    pltpu.sync_copy(idx_hbm.at[pl.ds(base, b_per_w)], idx_v)

    bufs = (buf0, buf1, buf2, buf3, buf4, buf5, buf6, buf7)
    nbuf = len(bufs)
    depth = nbuf - 1
    steps = [(p, c) for p in range(NUM_P) for c in range(n_chunks)]

    def gather_start(step, buf):
      p, c = step
      return pltpu.async_copy(
          table_hbm.at[p].at[idx_v.at[pl.ds(c * CHUNK, CHUNK)]], buf, gsem)

    def write_start(step, buf):
      p, c = step
      return pltpu.async_copy(
          buf, out_hbm.at[p].at[pl.ds(base + c * CHUNK, CHUNK)], wsem)

    def write_drain(step, buf):
      p, c = step
      pltpu.make_async_copy(
          buf, out_hbm.at[p].at[pl.ds(base + c * CHUNK, CHUNK)], wsem).wait()

    n = len(steps)
    gathers = [gather_start(steps[k], bufs[k % nbuf])
               for k in range(min(depth, n))]
    for k in range(n):
      gathers[k].wait()
      write_start(steps[k], bufs[k % nbuf])
      j = k + depth
      if j < n:
        if j - nbuf >= 0:
          write_drain(steps[j - nbuf], bufs[j % nbuf])
        gathers.append(gather_start(steps[j], bufs[j % nbuf]))
    for k in range(max(0, n - nbuf), n):
      write_drain(steps[k], bufs[k % nbuf])

  return gather_kernel


_gather = _make_gather()


@jax.jit
def kernel(sg_params, scene_id):
  table = jnp.transpose(sg_params, (2, 0, 1))
  out7 = _gather(table, scene_id.astype(jnp.int32))
  return jnp.transpose(out7, (1, 2, 0))


# speedup vs baseline: 1.0236x; 1.0236x over previous
import functools

import jax
import jax.numpy as jnp
from jax import lax
from jax.experimental import pallas as pl
from jax.experimental.pallas import tpu as pltpu
from jax.experimental.pallas import tpu_sc as plsc

NUM_SCENES = 100000
NUM_LOBES = 128
NUM_P = 7
BATCH = 16384
CHUNK = 128


def _make_gather():
  info = plsc.get_sparse_core_info()
  nw = info.num_cores * info.num_subcores
  b_per_w = BATCH // nw
  n_chunks = b_per_w // CHUNK

  mesh = plsc.VectorSubcoreMesh(core_axis_name="c", subcore_axis_name="s")

  @functools.partial(
      pl.kernel,
      mesh=mesh,
      out_type=jax.ShapeDtypeStruct((NUM_P, BATCH, NUM_LOBES), jnp.float32),
      scratch_types=[
          pltpu.VMEM((b_per_w,), jnp.int32),
          pltpu.VMEM((CHUNK, NUM_LOBES), jnp.float32),
          pltpu.VMEM((CHUNK, NUM_LOBES), jnp.float32),
          pltpu.VMEM((CHUNK, NUM_LOBES), jnp.float32),
          pltpu.VMEM((CHUNK, NUM_LOBES), jnp.float32),
          pltpu.SemaphoreType.DMA,
          pltpu.SemaphoreType.DMA,
      ],
  )
  def gather_kernel(table_hbm, idx_hbm, out_hbm, idx_v, buf0, buf1,
                    buf2, buf3, gsem, wsem):
    wid = lax.axis_index("s") * info.num_cores + lax.axis_index("c")
    base = wid * b_per_w
    pltpu.sync_copy(idx_hbm.at[pl.ds(base, b_per_w)], idx_v)

    bufs = (buf0, buf1, buf2, buf3)
    nbuf = len(bufs)
    depth = nbuf - 1
    steps = [(p, c) for p in range(NUM_P) for c in range(n_chunks)]

    def gather_start(step, buf):
      p, c = step
      return pltpu.async_copy(
          table_hbm.at[p].at[idx_v.at[pl.ds(c * CHUNK, CHUNK)]], buf, gsem)

    def write_start(step, buf):
      p, c = step
      return pltpu.async_copy(
          buf, out_hbm.at[p].at[pl.ds(base + c * CHUNK, CHUNK)], wsem)

    def write_drain(step, buf):
      p, c = step
      pltpu.make_async_copy(
          buf, out_hbm.at[p].at[pl.ds(base + c * CHUNK, CHUNK)], wsem).wait()

    n = len(steps)
    gathers = [gather_start(steps[k], bufs[k % nbuf])
               for k in range(min(depth, n))]
    for k in range(n):
      gathers[k].wait()
      write_start(steps[k], bufs[k % nbuf])
      j = k + depth
      if j < n:
        if j - nbuf >= 0:
          write_drain(steps[j - nbuf], bufs[j % nbuf])
        gathers.append(gather_start(steps[j], bufs[j % nbuf]))
    for k in range(max(0, n - nbuf), n):
      write_drain(steps[k], bufs[k % nbuf])

  return gather_kernel


_gather = _make_gather()


@jax.jit
def kernel(sg_params, scene_id):
  table = jnp.transpose(sg_params, (2, 0, 1))
  out7 = _gather(table, scene_id.astype(jnp.int32))
  return jnp.transpose(out7, (1, 2, 0))


# speedup vs baseline: 1.0260x; 1.0023x over previous
import functools

import jax
import jax.numpy as jnp
from jax import lax
from jax.experimental import pallas as pl
from jax.experimental.pallas import tpu as pltpu
from jax.experimental.pallas import tpu_sc as plsc

NUM_SCENES = 100000
NUM_LOBES = 128
NUM_P = 7
BATCH = 16384
CHUNK = 128


def _make_gather():
  info = plsc.get_sparse_core_info()
  nw = info.num_cores * info.num_subcores
  b_per_w = BATCH // nw
  n_chunks = b_per_w // CHUNK

  mesh = plsc.VectorSubcoreMesh(core_axis_name="c", subcore_axis_name="s")

  @functools.partial(
      pl.kernel,
      mesh=mesh,
      out_type=jax.ShapeDtypeStruct((NUM_P, BATCH, NUM_LOBES), jnp.float32),
      scratch_types=[
          pltpu.VMEM((b_per_w,), jnp.int32),
          pltpu.VMEM((CHUNK, NUM_LOBES), jnp.float32),
          pltpu.VMEM((CHUNK, NUM_LOBES), jnp.float32),
          pltpu.VMEM((CHUNK, NUM_LOBES), jnp.float32),
          pltpu.VMEM((CHUNK, NUM_LOBES), jnp.float32),
          pltpu.SemaphoreType.DMA,
          pltpu.SemaphoreType.DMA,
      ],
  )
  def gather_kernel(table_hbm, idx_hbm, out_hbm, idx_v, buf0, buf1,
                    buf2, buf3, gsem, wsem):
    wid = lax.axis_index("c") * info.num_subcores + lax.axis_index("s")
    base = wid * b_per_w
    pltpu.sync_copy(idx_hbm.at[pl.ds(base, b_per_w)], idx_v)

    bufs = (buf0, buf1, buf2, buf3)
    nbuf = len(bufs)
    depth = nbuf - 1
    steps = [(p, c) for p in range(NUM_P) for c in range(n_chunks)]

    def gather_start(step, buf):
      p, c = step
      return pltpu.async_copy(
          table_hbm.at[p].at[idx_v.at[pl.ds(c * CHUNK, CHUNK)]], buf, gsem)

    def write_start(step, buf):
      p, c = step
      return pltpu.async_copy(
          buf, out_hbm.at[p].at[pl.ds(base + c * CHUNK, CHUNK)], wsem)

    def write_drain(step, buf):
      p, c = step
      pltpu.make_async_copy(
          buf, out_hbm.at[p].at[pl.ds(base + c * CHUNK, CHUNK)], wsem).wait()

    n = len(steps)
    gathers = [gather_start(steps[k], bufs[k % nbuf])
               for k in range(min(depth, n))]
    for k in range(n):
      gathers[k].wait()
      write_start(steps[k], bufs[k % nbuf])
      j = k + depth
      if j < n:
        if j - nbuf >= 0:
          write_drain(steps[j - nbuf], bufs[j % nbuf])
        gathers.append(gather_start(steps[j], bufs[j % nbuf]))
    for k in range(max(0, n - nbuf), n):
      write_drain(steps[k], bufs[k % nbuf])

  return gather_kernel


_gather = _make_gather()


@jax.jit
def kernel(sg_params, scene_id):
  table = jnp.transpose(sg_params, (2, 0, 1))
  out7 = _gather(table, scene_id.astype(jnp.int32))
  return jnp.transpose(out7, (1, 2, 0))
